# baseline (speedup 1.0000x reference)
import jax
import jax.numpy as jnp
import numpy as np
from jax import lax
from jax.experimental import pallas as pl
from jax.experimental.pallas import tpu as pltpu
from jax.experimental.pallas import tpu_sc as plsc

N_NODES = 10000
N_EDGES = 320000
D_IN = 128
D_HID = 128
D_OUT = 64

NPAD = 10240
EPAD = 327680
CHUNK = 128
CHUNKS_PER_W = EPAD // (32 * CHUNK)
ROWS_PER_TILE = NPAD // 16
_GRP = 16
_NGRP = CHUNKS_PER_W // _GRP

_sc_mesh = plsc.VectorSubcoreMesh(core_axis_name="c", subcore_axis_name="s")


_DEG_GRP = 8


def _deg_body(edges_hbm, zeros_hbm, out_hbm, acc, idxv, ones, sem):
    c = lax.axis_index("c")
    s = lax.axis_index("s")
    wid = c * 16 + s
    base = s * ROWS_PER_TILE
    pltpu.sync_copy(zeros_hbm.at[pl.ds(base, ROWS_PER_TILE)],
                    acc.at[pl.ds(base, ROWS_PER_TILE)])
    for i in range(CHUNK // 16):
        ones[pl.ds(i * 16, 16)] = jnp.full((16,), 1.0, jnp.float32)
    pltpu.sync_copy(edges_hbm.at[1, pl.ds(wid * CHUNKS_PER_W, CHUNKS_PER_W)],
                    idxv)
    plsc.subcore_barrier()

    def group(g, carry):
        for j in range(_DEG_GRP):
            pltpu.async_copy(ones, acc.at[idxv.at[g * _DEG_GRP + j]], sem,
                             add=True)
        for j in range(_DEG_GRP):
            pltpu.make_async_copy(ones, acc.at[idxv.at[g * _DEG_GRP + j]],
                                  sem).wait()
        return carry

    lax.fori_loop(0, CHUNKS_PER_W // _DEG_GRP, group, 0)
    plsc.subcore_barrier()
    pltpu.sync_copy(acc.at[pl.ds(base, ROWS_PER_TILE)],
                    out_hbm.at[c, pl.ds(base, ROWS_PER_TILE)])


_deg_kernel = pl.kernel(
    _deg_body,
    out_type=jax.ShapeDtypeStruct((2, NPAD), jnp.float32),
    mesh=_sc_mesh,
    scratch_types=[
        pltpu.VMEM_SHARED((NPAD,), jnp.float32),
        pltpu.VMEM((CHUNKS_PER_W, CHUNK), jnp.int32),
        pltpu.VMEM((CHUNK,), jnp.float32),
        pltpu.SemaphoreType.DMA,
    ],
)


def _make_scat_body(nbuf, ring, ck, d, bf16_gather):
    cpw = EPAD // (32 * ck)
    ngrp = cpw // _GRP

    def body(table_hbm, edges_hbm, zeros_hbm, out_hbm,
             acc, srcv, dstv, rows, *rest):
        if bf16_gather:
            rowsf = rest[0]
            sems = rest[1:]
        else:
            rowsf = rows
            sems = rest
        gsems = sems[:nbuf]
        ssems = sems[nbuf:2 * nbuf]
        isems = sems[2 * nbuf:]
        c = lax.axis_index("c")
        s = lax.axis_index("s")
        wid = c * 16 + s
        base = s * ROWS_PER_TILE

        def idx_load(g, slot):
            grow = wid * cpw + g * _GRP
            pltpu.async_copy(edges_hbm.at[0, pl.ds(grow, _GRP)],
                             srcv.at[slot], isems[slot])
            pltpu.async_copy(edges_hbm.at[1, pl.ds(grow, _GRP)],
                             dstv.at[slot], isems[slot])

        def idx_wait(g, slot):
            grow = wid * cpw + g * _GRP
            pltpu.make_async_copy(edges_hbm.at[0, pl.ds(grow, _GRP)],
                                  srcv.at[slot], isems[slot]).wait()
            pltpu.make_async_copy(edges_hbm.at[1, pl.ds(grow, _GRP)],
                                  dstv.at[slot], isems[slot]).wait()

        def gat(slot, j, b):
            pltpu.async_copy(table_hbm.at[srcv.at[slot].at[j]], rows.at[b],
                             gsems[b])

        def wait_gat(slot, j, b):
            pltpu.make_async_copy(table_hbm.at[srcv.at[slot].at[j]],
                                  rows.at[b], gsems[b]).wait()

        def scat_sync(slot, j, b):
            pltpu.sync_copy(rowsf.at[b], acc.at[dstv.at[slot].at[j]], add=True)

        def scat(slot, j, b):
            pltpu.async_copy(rowsf.at[b], acc.at[dstv.at[slot].at[j]],
                             ssems[b], add=True)

        def wait_scat(slot, j, b):
            pltpu.make_async_copy(rowsf.at[b], acc.at[dstv.at[slot].at[j]],
                                  ssems[b]).wait()

        def unpack_chunk(b):
            if not bf16_gather:
                return
            bf = rows.at[b]
            ff = rowsf.at[b]

            def row2(r, carry):
                for rr in range(2):
                    for k2 in range(d // 32):
                        v = bf[2 * r + rr, pl.ds(32 * k2, 32)]
                        lo, hi = plsc.unpack(
                            v, format=plsc.PackFormat.INTERLEAVED,
                            preferred_element_type=jnp.float32)
                        ff[2 * r + rr, pl.ds(32 * k2, 16)] = lo
                        ff[2 * r + rr, pl.ds(32 * k2 + 16, 16)] = hi
                return carry

            lax.fori_loop(0, ck // 2, row2, 0)

        idx_load(0, 0)
        pltpu.sync_copy(zeros_hbm.at[pl.ds(base, ROWS_PER_TILE)],
                        acc.at[pl.ds(base, ROWS_PER_TILE)])
        plsc.subcore_barrier()

        for g in range(ngrp):
            slot = g % 2
            idx_wait(g, slot)
            if g + 1 < ngrp:
                idx_load(g + 1, 1 - slot)

            if not ring:
                gat(slot, 0, 0)

                def pair(k, carry2, slot=slot):
                    j0 = 2 * k
                    gat(slot, j0 + 1, 1)
                    wait_gat(slot, j0, 0)
                    scat_sync(slot, j0, 0)
                    gat(slot, j0 + 2, 0)
                    wait_gat(slot, j0 + 1, 1)
                    scat_sync(slot, j0 + 1, 1)
                    return carry2

                lax.fori_loop(0, _GRP // 2 - 1, pair, 0)
                j0 = _GRP - 2
                gat(slot, j0 + 1, 1)
                wait_gat(slot, j0, 0)
                scat_sync(slot, j0, 0)
                wait_gat(slot, j0 + 1, 1)
                scat_sync(slot, j0 + 1, 1)
            else:
                for b in range(nbuf):
                    gat(slot, b, b)

                def rnd(r, carry2, slot=slot):
                    for b in range(nbuf):
                        j = r * nbuf + b
                        wait_gat(slot, j, b)
                        unpack_chunk(b)
                        scat(slot, j, b)
                    for b in range(nbuf):
                        j = r * nbuf + b
                        wait_scat(slot, j, b)
                        gat(slot, j + nbuf, b)
                    return carry2

                lax.fori_loop(0, _GRP // nbuf - 1, rnd, 0)
                jl = _GRP - nbuf
                for b in range(nbuf):
                    wait_gat(slot, jl + b, b)
                    unpack_chunk(b)
                    scat(slot, jl + b, b)
                for b in range(nbuf):
                    wait_scat(slot, jl + b, b)

        plsc.subcore_barrier()
        pltpu.sync_copy(acc.at[pl.ds(base, ROWS_PER_TILE)],
                        out_hbm.at[c, pl.ds(base, ROWS_PER_TILE)])

    return body


def _make_scat_kernel(d, nbuf, ring, tc_tiling, ck, bf16_gather=False):
    scratch = [
        pltpu.VMEM_SHARED((NPAD, d), jnp.float32),
        pltpu.VMEM((2, _GRP, ck), jnp.int32),
        pltpu.VMEM((2, _GRP, ck), jnp.int32),
    ]
    if bf16_gather:
        scratch += [pltpu.VMEM((nbuf, ck, d), jnp.bfloat16),
                    pltpu.VMEM((nbuf, ck, d), jnp.float32)]
    else:
        scratch += [pltpu.VMEM((nbuf, ck, d), jnp.float32)]
    return pl.kernel(
        _make_scat_body(nbuf, ring, ck, d, bf16_gather),
        out_type=jax.ShapeDtypeStruct((2, NPAD, d), jnp.float32),
        mesh=_sc_mesh,
        compiler_params=pltpu.CompilerParams(
            use_tc_tiling_on_sc=tc_tiling,
            needs_layout_passes=not bf16_gather),
        scratch_types=scratch + [pltpu.SemaphoreType.DMA] * (2 * nbuf + 2),
    )


_CK1 = 128
_scat_hid = _make_scat_kernel(D_HID, 2, False, True, _CK1)
_scat_out = _make_scat_kernel(D_OUT, 4, True, False, CHUNK, bf16_gather=True)


def _tc1_body(x_ref, w0_ref, w1_ref, degt_ref, g1_ref, dinv_ref):
    d = degt_ref[:, 0:1] + degt_ref[:, 1:2] + 1.0
    dinv = lax.rsqrt(d)
    w01 = jnp.dot(w0_ref[...], w1_ref[...], preferred_element_type=jnp.float32)
    h = jnp.dot(x_ref[...], w01, preferred_element_type=jnp.float32)
    g1_ref[...] = h * dinv
    dinv_ref[...] = dinv


def _tc2_body(p0_ref, p1_ref, g1_ref, dinv_ref, b1_ref, w2_ref,
              g2_ref, g2b_ref):
    dinv = dinv_ref[...]
    agg = (p0_ref[0] + p1_ref[0] + g1_ref[...]) * dinv + b1_ref[...]
    h = jnp.maximum(agg, 0.0)
    g2 = jnp.dot(h, w2_ref[...], preferred_element_type=jnp.float32) * dinv
    g2_ref[...] = g2
    t = g2.reshape(_BLK, D_OUT // 32, 2, 16)
    g2b_ref[...] = jnp.swapaxes(t, 2, 3).reshape(_BLK, D_OUT).astype(
        jnp.bfloat16)


def _tc3_body(q0_ref, q1_ref, g2_ref, dinv_ref, b2_ref, out_ref):
    out_ref[...] = ((q0_ref[0] + q1_ref[0] + g2_ref[...]) * dinv_ref[...]
                    + b2_ref[...])


_BLK = 1024
_NBLK = NPAD // _BLK


def _row_spec(d):
    return pl.BlockSpec((_BLK, d), lambda i: (i, 0))


def _part_spec(part, d):
    return pl.BlockSpec((1, _BLK, d), lambda i, _p=part: (_p, i, 0))


def _full_spec(r, c):
    return pl.BlockSpec((r, c), lambda i: (0, 0))


_tc1_call = pl.pallas_call(
    _tc1_body,
    grid=(_NBLK,),
    in_specs=[_row_spec(D_IN), _full_spec(D_IN, D_HID),
              _full_spec(D_HID, D_HID), _row_spec(2)],
    out_specs=[_row_spec(D_HID), _row_spec(1)],
    out_shape=[jax.ShapeDtypeStruct((NPAD, D_HID), jnp.float32),
               jax.ShapeDtypeStruct((NPAD, 1), jnp.float32)],
)

_tc2_call = pl.pallas_call(
    _tc2_body,
    grid=(_NBLK,),
    in_specs=[_part_spec(0, D_HID), _part_spec(1, D_HID), _row_spec(D_HID),
              _row_spec(1), _full_spec(1, D_HID), _full_spec(D_HID, D_OUT)],
    out_specs=[_row_spec(D_OUT), _row_spec(D_OUT)],
    out_shape=[jax.ShapeDtypeStruct((NPAD, D_OUT), jnp.float32),
               jax.ShapeDtypeStruct((NPAD, D_OUT), jnp.bfloat16)],
)

_tc3_call = pl.pallas_call(
    _tc3_body,
    grid=(_NBLK,),
    in_specs=[_part_spec(0, D_OUT), _part_spec(1, D_OUT), _row_spec(D_OUT),
              _row_spec(1), _full_spec(1, D_OUT)],
    out_specs=_row_spec(D_OUT),
    out_shape=jax.ShapeDtypeStruct((N_NODES, D_OUT), jnp.float32),
)

_PAD_EDGES = np.tile(N_NODES + np.arange(EPAD - N_EDGES, dtype=np.int32)
                     % (NPAD - N_NODES), (2, 1))


@jax.jit
def kernel(x, edge_index, W0, W1, b1, W2, b2):
    edges = jnp.concatenate([edge_index, jnp.asarray(_PAD_EDGES)], axis=1)
    edges128 = edges.reshape(2, EPAD // CHUNK, CHUNK)
    edges80 = edges.reshape(2, EPAD // _CK1, _CK1)
    zeros1 = jnp.zeros((NPAD,), jnp.float32)
    zeros_h = jnp.zeros((NPAD, D_HID), jnp.float32)
    zeros_o = jnp.zeros((NPAD, D_OUT), jnp.float32)

    degp = _deg_kernel(edges128, zeros1)
    degt = degp.T

    g1, dinv = _tc1_call(x, W0, W1, degt)

    p = _scat_hid(g1, edges80, zeros_h)

    g2, g2b = _tc2_call(p, p, g1, dinv, b1[None, :], W2)

    q = _scat_out(g2b, edges128, zeros_o)

    return _tc3_call(q, q, g2, dinv, b2[None, :])


# speedup vs baseline: 1.5039x; 1.5039x over previous
import jax
import jax.numpy as jnp
import numpy as np
from jax import lax
from jax.experimental import pallas as pl
from jax.experimental.pallas import tpu as pltpu
from jax.experimental.pallas import tpu_sc as plsc

N_NODES = 10000
N_EDGES = 320000
D_IN = 128
D_HID = 128
D_OUT = 64

NPAD = 10240
EPAD = 327680
CHUNK = 128
CHUNKS_PER_W = EPAD // (32 * CHUNK)
ROWS_PER_TILE = NPAD // 16
_GRP = 16
_NGRP = CHUNKS_PER_W // _GRP

_sc_mesh = plsc.VectorSubcoreMesh(core_axis_name="c", subcore_axis_name="s")


_DEG_GRP = 8


def _deg_body(edges_hbm, zeros_hbm, out_hbm, acc, idxv, ones, sem):
    c = lax.axis_index("c")
    s = lax.axis_index("s")
    wid = c * 16 + s
    base = s * ROWS_PER_TILE
    pltpu.sync_copy(zeros_hbm.at[pl.ds(base, ROWS_PER_TILE)],
                    acc.at[pl.ds(base, ROWS_PER_TILE)])
    for i in range(CHUNK // 16):
        ones[pl.ds(i * 16, 16)] = jnp.full((16,), 1.0, jnp.float32)
    pltpu.sync_copy(edges_hbm.at[1, pl.ds(wid * CHUNKS_PER_W, CHUNKS_PER_W)],
                    idxv)
    plsc.subcore_barrier()

    def group(g, carry):
        for j in range(_DEG_GRP):
            pltpu.async_copy(ones, acc.at[idxv.at[g * _DEG_GRP + j]], sem,
                             add=True)
        for j in range(_DEG_GRP):
            pltpu.make_async_copy(ones, acc.at[idxv.at[g * _DEG_GRP + j]],
                                  sem).wait()
        return carry

    lax.fori_loop(0, CHUNKS_PER_W // _DEG_GRP, group, 0)
    plsc.subcore_barrier()
    pltpu.sync_copy(acc.at[pl.ds(base, ROWS_PER_TILE)],
                    out_hbm.at[c, pl.ds(base, ROWS_PER_TILE)])


_deg_kernel = pl.kernel(
    _deg_body,
    out_type=jax.ShapeDtypeStruct((2, NPAD), jnp.float32),
    mesh=_sc_mesh,
    scratch_types=[
        pltpu.VMEM_SHARED((NPAD,), jnp.float32),
        pltpu.VMEM((CHUNKS_PER_W, CHUNK), jnp.int32),
        pltpu.VMEM((CHUNK,), jnp.float32),
        pltpu.SemaphoreType.DMA,
    ],
)


def _make_scat_body(nbuf, ring, ck):
    cpw = EPAD // (32 * ck)
    ngrp = cpw // _GRP

    def body(table_hbm, edges_hbm, zeros_hbm, out_hbm,
             acc, srcv, dstv, rows, *sems):
        gsems = sems[:nbuf]
        ssems = sems[nbuf:2 * nbuf]
        isems = sems[2 * nbuf:]
        c = lax.axis_index("c")
        s = lax.axis_index("s")
        wid = c * 16 + s
        base = s * ROWS_PER_TILE

        def idx_load(g, slot):
            grow = wid * cpw + g * _GRP
            pltpu.async_copy(edges_hbm.at[0, pl.ds(grow, _GRP)],
                             srcv.at[slot], isems[slot])
            pltpu.async_copy(edges_hbm.at[1, pl.ds(grow, _GRP)],
                             dstv.at[slot], isems[slot])

        def idx_wait(g, slot):
            grow = wid * cpw + g * _GRP
            pltpu.make_async_copy(edges_hbm.at[0, pl.ds(grow, _GRP)],
                                  srcv.at[slot], isems[slot]).wait()
            pltpu.make_async_copy(edges_hbm.at[1, pl.ds(grow, _GRP)],
                                  dstv.at[slot], isems[slot]).wait()

        def gat(slot, j, b):
            pltpu.async_copy(table_hbm.at[srcv.at[slot].at[j]], rows.at[b],
                             gsems[b])

        def wait_gat(slot, j, b):
            pltpu.make_async_copy(table_hbm.at[srcv.at[slot].at[j]],
                                  rows.at[b], gsems[b]).wait()

        def scat_sync(slot, j, b):
            pltpu.sync_copy(rows.at[b], acc.at[dstv.at[slot].at[j]], add=True)

        def scat(slot, j, b):
            pltpu.async_copy(rows.at[b], acc.at[dstv.at[slot].at[j]],
                             ssems[b], add=True)

        def wait_scat(slot, j, b):
            pltpu.make_async_copy(rows.at[b], acc.at[dstv.at[slot].at[j]],
                                  ssems[b]).wait()

        idx_load(0, 0)
        pltpu.sync_copy(zeros_hbm.at[pl.ds(base, ROWS_PER_TILE)],
                        acc.at[pl.ds(base, ROWS_PER_TILE)])
        plsc.subcore_barrier()

        for g in range(ngrp):
            slot = g % 2
            idx_wait(g, slot)
            if g + 1 < ngrp:
                idx_load(g + 1, 1 - slot)

            if not ring:
                gat(slot, 0, 0)

                def pair(k, carry2, slot=slot):
                    j0 = 2 * k
                    gat(slot, j0 + 1, 1)
                    wait_gat(slot, j0, 0)
                    scat_sync(slot, j0, 0)
                    gat(slot, j0 + 2, 0)
                    wait_gat(slot, j0 + 1, 1)
                    scat_sync(slot, j0 + 1, 1)
                    return carry2

                lax.fori_loop(0, _GRP // 2 - 1, pair, 0)
                j0 = _GRP - 2
                gat(slot, j0 + 1, 1)
                wait_gat(slot, j0, 0)
                scat_sync(slot, j0, 0)
                wait_gat(slot, j0 + 1, 1)
                scat_sync(slot, j0 + 1, 1)
            else:
                for b in range(nbuf):
                    gat(slot, b, b)

                def rnd(r, carry2, slot=slot):
                    for b in range(nbuf):
                        j = r * nbuf + b
                        wait_gat(slot, j, b)
                        scat(slot, j, b)
                    for b in range(nbuf):
                        j = r * nbuf + b
                        wait_scat(slot, j, b)
                        gat(slot, j + nbuf, b)
                    return carry2

                lax.fori_loop(0, _GRP // nbuf - 1, rnd, 0)
                jl = _GRP - nbuf
                for b in range(nbuf):
                    wait_gat(slot, jl + b, b)
                    scat(slot, jl + b, b)
                for b in range(nbuf):
                    wait_scat(slot, jl + b, b)

        plsc.subcore_barrier()
        pltpu.sync_copy(acc.at[pl.ds(base, ROWS_PER_TILE)],
                        out_hbm.at[c, pl.ds(base, ROWS_PER_TILE)])

    return body


def _make_scat_kernel(d, nbuf, ring, tc_tiling, ck):
    return pl.kernel(
        _make_scat_body(nbuf, ring, ck),
        out_type=jax.ShapeDtypeStruct((2, NPAD, d), jnp.float32),
        mesh=_sc_mesh,
        compiler_params=pltpu.CompilerParams(use_tc_tiling_on_sc=tc_tiling),
        scratch_types=[
            pltpu.VMEM_SHARED((NPAD, d), jnp.float32),
            pltpu.VMEM((2, _GRP, ck), jnp.int32),
            pltpu.VMEM((2, _GRP, ck), jnp.int32),
            pltpu.VMEM((nbuf, ck, d), jnp.float32),
        ] + [pltpu.SemaphoreType.DMA] * (2 * nbuf + 2),
    )


_CK1 = 128
_scat_hid = _make_scat_kernel(D_HID, 2, False, True, _CK1)
_scat_out = _make_scat_kernel(D_OUT, 8, True, False, CHUNK)


def _tc1_body(x_ref, w0_ref, w1_ref, degt_ref, g1_ref, dinv_ref):
    d = degt_ref[:, 0:1] + degt_ref[:, 1:2] + 1.0
    dinv = lax.rsqrt(d)
    w01 = jnp.dot(w0_ref[...], w1_ref[...], preferred_element_type=jnp.float32)
    h = jnp.dot(x_ref[...], w01, preferred_element_type=jnp.float32)
    g1_ref[...] = h * dinv
    dinv_ref[...] = dinv


def _tc2_body(p0_ref, p1_ref, g1_ref, dinv_ref, b1_ref, w2_ref, g2_ref):
    dinv = dinv_ref[...]
    agg = (p0_ref[0] + p1_ref[0] + g1_ref[...]) * dinv + b1_ref[...]
    h = jnp.maximum(agg, 0.0)
    g2_ref[...] = jnp.dot(h, w2_ref[...],
                          preferred_element_type=jnp.float32) * dinv


def _tc3_body(q0_ref, q1_ref, g2_ref, dinv_ref, b2_ref, out_ref):
    out_ref[...] = ((q0_ref[0] + q1_ref[0] + g2_ref[...]) * dinv_ref[...]
                    + b2_ref[...])


_BLK = 1024
_NBLK = NPAD // _BLK


def _row_spec(d):
    return pl.BlockSpec((_BLK, d), lambda i: (i, 0))


def _part_spec(part, d):
    return pl.BlockSpec((1, _BLK, d), lambda i, _p=part: (_p, i, 0))


def _full_spec(r, c):
    return pl.BlockSpec((r, c), lambda i: (0, 0))


_tc1_call = pl.pallas_call(
    _tc1_body,
    grid=(_NBLK,),
    in_specs=[_row_spec(D_IN), _full_spec(D_IN, D_HID),
              _full_spec(D_HID, D_HID), _row_spec(2)],
    out_specs=[_row_spec(D_HID), _row_spec(1)],
    out_shape=[jax.ShapeDtypeStruct((NPAD, D_HID), jnp.float32),
               jax.ShapeDtypeStruct((NPAD, 1), jnp.float32)],
)

_tc2_call = pl.pallas_call(
    _tc2_body,
    grid=(_NBLK,),
    in_specs=[_part_spec(0, D_HID), _part_spec(1, D_HID), _row_spec(D_HID),
              _row_spec(1), _full_spec(1, D_HID), _full_spec(D_HID, D_OUT)],
    out_specs=_row_spec(D_OUT),
    out_shape=jax.ShapeDtypeStruct((NPAD, D_OUT), jnp.float32),
)

_tc3_call = pl.pallas_call(
    _tc3_body,
    grid=(_NBLK,),
    in_specs=[_part_spec(0, D_OUT), _part_spec(1, D_OUT), _row_spec(D_OUT),
              _row_spec(1), _full_spec(1, D_OUT)],
    out_specs=_row_spec(D_OUT),
    out_shape=jax.ShapeDtypeStruct((N_NODES, D_OUT), jnp.float32),
)

_PAD_EDGES = np.tile(N_NODES + np.arange(EPAD - N_EDGES, dtype=np.int32)
                     % (NPAD - N_NODES), (2, 1))


@jax.jit
def kernel(x, edge_index, W0, W1, b1, W2, b2):
    edges = jnp.concatenate([edge_index, jnp.asarray(_PAD_EDGES)], axis=1)
    edges128 = edges.reshape(2, EPAD // CHUNK, CHUNK)
    edges80 = edges.reshape(2, EPAD // _CK1, _CK1)
    zeros1 = jnp.zeros((NPAD,), jnp.float32)
    zeros_h = jnp.zeros((NPAD, D_HID), jnp.float32)
    zeros_o = jnp.zeros((NPAD, D_OUT), jnp.float32)

    degp = _deg_kernel(edges128, zeros1)
    degt = degp.T

    g1, dinv = _tc1_call(x, W0, W1, degt)

    p = _scat_hid(g1, edges80, zeros_h)

    g2 = _tc2_call(p, p, g1, dinv, b1[None, :], W2)

    q = _scat_out(g2, edges128, zeros_o)

    return _tc3_call(q, q, g2, dinv, b2[None, :])
